# baseline (speedup 1.0000x reference)
import jax
import jax.numpy as jnp
from jax import lax
from jax.experimental import pallas as pl
from jax.experimental.pallas import tpu as pltpu
from jax.experimental.pallas import tpu_sc as plsc

S = 1024
R = 65536

_NC = 2
_NS = 16
_LANES = 16
_NW = _NC * _NS
_CHUNK = R // _NW


def _sc_rates_body(ab_hbm, idx0_hbm, idx1_hbm, al_hbm, be_hbm, ga_hbm,
                   ac_hbm, af_hbm, sc_hbm, out_hbm,
                   ab_v, i0_v, i1_v, al_v, be_v, ga_v, ac_v, af_v, sc_v, f_v):
    wid = lax.axis_index("s") * _NC + lax.axis_index("c")
    base = wid * _CHUNK
    pltpu.sync_copy(ab_hbm, ab_v)
    pltpu.sync_copy(sc_hbm, sc_v)
    pltpu.sync_copy(idx0_hbm.at[pl.ds(base, _CHUNK)], i0_v)
    pltpu.sync_copy(idx1_hbm.at[pl.ds(base, _CHUNK)], i1_v)
    pltpu.sync_copy(al_hbm.at[pl.ds(base, _CHUNK)], al_v)
    pltpu.sync_copy(be_hbm.at[pl.ds(base, _CHUNK)], be_v)
    pltpu.sync_copy(ga_hbm.at[pl.ds(base, _CHUNK)], ga_v)
    pltpu.sync_copy(ac_hbm.at[pl.ds(base, _CHUNK)], ac_v)
    pltpu.sync_copy(af_hbm.at[pl.ds(base, _CHUNK)], af_v)

    ltv = sc_v[pl.ds(0, _LANES)]
    niv = sc_v[pl.ds(16, _LANES)]
    crv = sc_v[pl.ds(32, _LANES)]
    fuvv = sc_v[pl.ds(48, _LANES)]

    def step(i, carry):
        off = i * _LANES
        sl = pl.ds(off, _LANES)
        a0 = plsc.load_gather(ab_v, [i0_v[sl]])
        a1 = plsc.load_gather(ab_v, [i1_v[sl]])
        rate = (al_v[sl] * jnp.exp(be_v[sl] * ltv + ga_v[sl] * niv)
                + ac_v[sl] * crv + af_v[sl] * fuvv)
        f_v[sl] = rate * a0 * a1
        return carry

    lax.fori_loop(0, _CHUNK // _LANES, step, 0, unroll=4)
    pltpu.sync_copy(f_v, out_hbm.at[pl.ds(base, _CHUNK)])


_sc_rates = pl.kernel(
    _sc_rates_body,
    out_type=jax.ShapeDtypeStruct((R,), jnp.float32),
    mesh=plsc.VectorSubcoreMesh(
        core_axis_name="c", subcore_axis_name="s", num_cores=_NC,
        num_subcores=_NS),
    scratch_types=[
        pltpu.VMEM((S,), jnp.float32),
        pltpu.VMEM((_CHUNK,), jnp.int32),
        pltpu.VMEM((_CHUNK,), jnp.int32),
        pltpu.VMEM((_CHUNK,), jnp.float32),
        pltpu.VMEM((_CHUNK,), jnp.float32),
        pltpu.VMEM((_CHUNK,), jnp.float32),
        pltpu.VMEM((_CHUNK,), jnp.float32),
        pltpu.VMEM((_CHUNK,), jnp.float32),
        pltpu.VMEM((64,), jnp.float32),
        pltpu.VMEM((_CHUNK,), jnp.float32),
    ],
    compiler_params=pltpu.CompilerParams(needs_layout_passes=False),
)


_SSC = 256
_STC = S - _SSC
_ROWS_W = _SSC // _NW
_CH = 4096

_SB = 64
_NSTEPS = _STC // _SB


def _sc_mv_body(inc_hbm, v_hbm, out_hbm, inc_v, v_v, st_v):
    wid = lax.axis_index("s") * _NC + lax.axis_index("c")
    row0 = _STC + wid * _ROWS_W

    def chunk(c, accs):
        off = c * _CH
        pltpu.sync_copy(v_hbm.at[pl.ds(off, _CH)], v_v)
        pltpu.sync_copy(inc_hbm.at[pl.ds(row0, _ROWS_W), pl.ds(off, _CH)],
                        inc_v)

        def jstep(j, a):
            sl = pl.ds(j * _LANES, _LANES)
            v16 = v_v[sl]
            return tuple(a[k] + inc_v[k, sl] * v16 for k in range(_ROWS_W))

        return lax.fori_loop(0, _CH // _LANES, jstep, accs, unroll=2)

    zeros = jnp.zeros((_LANES,), jnp.float32)
    accs = lax.fori_loop(0, R // _CH, chunk, (zeros,) * _ROWS_W)
    for k in range(_ROWS_W):
        st_v[pl.ds(k * _LANES, _LANES)] = accs[k]
    pltpu.sync_copy(st_v, out_hbm.at[pl.ds(wid * _ROWS_W * _LANES,
                                           _ROWS_W * _LANES)])


_sc_mv = pl.kernel(
    _sc_mv_body,
    out_type=jax.ShapeDtypeStruct((_SSC * _LANES,), jnp.float32),
    mesh=plsc.VectorSubcoreMesh(
        core_axis_name="c", subcore_axis_name="s", num_cores=_NC,
        num_subcores=_NS),
    scratch_types=[
        pltpu.VMEM((_ROWS_W, _CH), jnp.float32),
        pltpu.VMEM((_CH,), jnp.float32),
        pltpu.VMEM((_ROWS_W * _LANES,), jnp.float32),
    ],
    compiler_params=pltpu.CompilerParams(needs_layout_passes=False),
)


def _tc_matvec_body(inc_ref, v_ref, out_ref):
    out_ref[...] = lax.dot_general(
        inc_ref[...], v_ref[...],
        dimension_numbers=(((1,), (1,)), ((), ())),
        preferred_element_type=jnp.float32)


_tc_matvec = pl.pallas_call(
    _tc_matvec_body,
    grid=(_NSTEPS,),
    in_specs=[
        pl.BlockSpec((_SB, R), lambda i: (i, 0)),
        pl.BlockSpec((1, R), lambda i: (0, 0)),
    ],
    out_specs=pl.BlockSpec((_SB, 1), lambda i: (i, 0)),
    out_shape=jax.ShapeDtypeStruct((_STC, 1), jnp.float32),
    compiler_params=pltpu.CompilerParams(
        dimension_semantics=("arbitrary",)),
)


def kernel(time, abundances, temperature, cr_rate, fuv_rate, incidence,
           alpha, beta, gamma, alpha_cr, alpha_fuv, species_idx):
    idx2 = species_idx.reshape(R, 2)
    idx0 = idx2[:, 0]
    idx1 = idx2[:, 1]

    scal = jnp.concatenate([
        jnp.full((16,), jnp.log(temperature / 300.0), dtype=jnp.float32),
        jnp.full((16,), -1.0 / temperature, dtype=jnp.float32),
        jnp.full((16,), cr_rate, dtype=jnp.float32),
        jnp.full((16,), fuv_rate, dtype=jnp.float32),
    ])

    v = _sc_rates(abundances, idx0, idx1, alpha, beta, gamma,
                  alpha_cr, alpha_fuv, scal)

    out_tc = _tc_matvec(incidence, v.reshape(1, R))
    out_sc = _sc_mv(incidence, v)
    return jnp.concatenate([
        out_tc.reshape(_STC),
        out_sc.reshape(_SSC, _LANES).sum(axis=1),
    ])


# speedup vs baseline: 1.0383x; 1.0383x over previous
import jax
import jax.numpy as jnp
from jax import lax
from jax.experimental import pallas as pl
from jax.experimental.pallas import tpu as pltpu
from jax.experimental.pallas import tpu_sc as plsc

S = 1024
R = 65536

_NC = 2
_NS = 16
_LANES = 16
_NW = _NC * _NS
_CHUNK = R // _NW


def _sc_rates_body(ab_hbm, idx0_hbm, idx1_hbm, al_hbm, be_hbm, ga_hbm,
                   ac_hbm, af_hbm, sc_hbm, out_hbm,
                   ab_v, i0_v, i1_v, al_v, be_v, ga_v, ac_v, af_v, sc_v, f_v):
    wid = lax.axis_index("s") * _NC + lax.axis_index("c")
    base = wid * _CHUNK
    pltpu.sync_copy(ab_hbm, ab_v)
    pltpu.sync_copy(sc_hbm, sc_v)
    pltpu.sync_copy(idx0_hbm.at[pl.ds(base, _CHUNK)], i0_v)
    pltpu.sync_copy(idx1_hbm.at[pl.ds(base, _CHUNK)], i1_v)
    pltpu.sync_copy(al_hbm.at[pl.ds(base, _CHUNK)], al_v)
    pltpu.sync_copy(be_hbm.at[pl.ds(base, _CHUNK)], be_v)
    pltpu.sync_copy(ga_hbm.at[pl.ds(base, _CHUNK)], ga_v)
    pltpu.sync_copy(ac_hbm.at[pl.ds(base, _CHUNK)], ac_v)
    pltpu.sync_copy(af_hbm.at[pl.ds(base, _CHUNK)], af_v)

    ltv = sc_v[pl.ds(0, _LANES)]
    niv = sc_v[pl.ds(16, _LANES)]
    crv = sc_v[pl.ds(32, _LANES)]
    fuvv = sc_v[pl.ds(48, _LANES)]

    def step(i, carry):
        off = i * _LANES
        sl = pl.ds(off, _LANES)
        a0 = plsc.load_gather(ab_v, [i0_v[sl]])
        a1 = plsc.load_gather(ab_v, [i1_v[sl]])
        rate = (al_v[sl] * jnp.exp(be_v[sl] * ltv + ga_v[sl] * niv)
                + ac_v[sl] * crv + af_v[sl] * fuvv)
        f_v[sl] = rate * a0 * a1
        return carry

    lax.fori_loop(0, _CHUNK // _LANES, step, 0, unroll=4)
    pltpu.sync_copy(f_v, out_hbm.at[pl.ds(base, _CHUNK)])


_sc_rates = pl.kernel(
    _sc_rates_body,
    out_type=jax.ShapeDtypeStruct((R,), jnp.float32),
    mesh=plsc.VectorSubcoreMesh(
        core_axis_name="c", subcore_axis_name="s", num_cores=_NC,
        num_subcores=_NS),
    scratch_types=[
        pltpu.VMEM((S,), jnp.float32),
        pltpu.VMEM((_CHUNK,), jnp.int32),
        pltpu.VMEM((_CHUNK,), jnp.int32),
        pltpu.VMEM((_CHUNK,), jnp.float32),
        pltpu.VMEM((_CHUNK,), jnp.float32),
        pltpu.VMEM((_CHUNK,), jnp.float32),
        pltpu.VMEM((_CHUNK,), jnp.float32),
        pltpu.VMEM((_CHUNK,), jnp.float32),
        pltpu.VMEM((64,), jnp.float32),
        pltpu.VMEM((_CHUNK,), jnp.float32),
    ],
    compiler_params=pltpu.CompilerParams(needs_layout_passes=False),
)


_SSC = 256
_STC = S - _SSC
_ROWS_W = _SSC // _NW
_CH = 4096

_SB = 64
_NSTEPS = _STC // _SB


def _sc_mv_body(inc_hbm, v_hbm, out_hbm, inc_v, v_v, st_v,
                sv0, sv1, si0, si1):
    wid = lax.axis_index("s") * _NC + lax.axis_index("c")
    row0 = _STC + wid * _ROWS_W
    nchunks = R // _CH
    svs = (sv0, sv1)
    sis = (si0, si1)

    def start(c, buf):
        off = c * _CH
        dv = pltpu.async_copy(v_hbm.at[pl.ds(off, _CH)], v_v.at[buf],
                              svs[buf])
        di = pltpu.async_copy(
            inc_hbm.at[pl.ds(row0, _ROWS_W), pl.ds(off, _CH)],
            inc_v.at[buf], sis[buf])
        return dv, di

    pend = start(0, 0)
    accs = (jnp.zeros((_LANES,), jnp.float32),) * _ROWS_W
    for c in range(nchunks):
        buf = c % 2
        pend[0].wait()
        pend[1].wait()
        if c + 1 < nchunks:
            pend = start(c + 1, 1 - buf)

        def jstep(j, a, buf=buf):
            sl = pl.ds(j * _LANES, _LANES)
            v16 = v_v[buf, sl]
            return tuple(a[k] + inc_v[buf, k, sl] * v16
                         for k in range(_ROWS_W))

        accs = lax.fori_loop(0, _CH // _LANES, jstep, accs, unroll=4)

    for k in range(_ROWS_W):
        st_v[pl.ds(k * _LANES, _LANES)] = accs[k]
    pltpu.sync_copy(st_v, out_hbm.at[pl.ds(wid * _ROWS_W * _LANES,
                                           _ROWS_W * _LANES)])


_sc_mv = pl.kernel(
    _sc_mv_body,
    out_type=jax.ShapeDtypeStruct((_SSC * _LANES,), jnp.float32),
    mesh=plsc.VectorSubcoreMesh(
        core_axis_name="c", subcore_axis_name="s", num_cores=_NC,
        num_subcores=_NS),
    scratch_types=[
        pltpu.VMEM((2, _ROWS_W, _CH), jnp.float32),
        pltpu.VMEM((2, _CH), jnp.float32),
        pltpu.VMEM((_ROWS_W * _LANES,), jnp.float32),
        pltpu.SemaphoreType.DMA,
        pltpu.SemaphoreType.DMA,
        pltpu.SemaphoreType.DMA,
        pltpu.SemaphoreType.DMA,
    ],
    compiler_params=pltpu.CompilerParams(needs_layout_passes=False),
)


def _tc_matvec_body(inc_ref, v_ref, out_ref):
    out_ref[...] = lax.dot_general(
        inc_ref[...], v_ref[...],
        dimension_numbers=(((1,), (1,)), ((), ())),
        preferred_element_type=jnp.float32)


_tc_matvec = pl.pallas_call(
    _tc_matvec_body,
    grid=(_NSTEPS,),
    in_specs=[
        pl.BlockSpec((_SB, R), lambda i: (i, 0)),
        pl.BlockSpec((1, R), lambda i: (0, 0)),
    ],
    out_specs=pl.BlockSpec((_SB, 1), lambda i: (i, 0)),
    out_shape=jax.ShapeDtypeStruct((_STC, 1), jnp.float32),
    compiler_params=pltpu.CompilerParams(
        dimension_semantics=("arbitrary",)),
)


def kernel(time, abundances, temperature, cr_rate, fuv_rate, incidence,
           alpha, beta, gamma, alpha_cr, alpha_fuv, species_idx):
    idx2 = species_idx.reshape(R, 2)
    idx0 = idx2[:, 0]
    idx1 = idx2[:, 1]

    scal = jnp.concatenate([
        jnp.full((16,), jnp.log(temperature / 300.0), dtype=jnp.float32),
        jnp.full((16,), -1.0 / temperature, dtype=jnp.float32),
        jnp.full((16,), cr_rate, dtype=jnp.float32),
        jnp.full((16,), fuv_rate, dtype=jnp.float32),
    ])

    v = _sc_rates(abundances, idx0, idx1, alpha, beta, gamma,
                  alpha_cr, alpha_fuv, scal)

    out_tc = _tc_matvec(incidence, v.reshape(1, R))
    out_sc = _sc_mv(incidence, v)
    return jnp.concatenate([
        out_tc.reshape(_STC),
        out_sc.reshape(_SSC, _LANES).sum(axis=1),
    ])


# speedup vs baseline: 1.0401x; 1.0018x over previous
import jax
import jax.numpy as jnp
from jax import lax
from jax.experimental import pallas as pl
from jax.experimental.pallas import tpu as pltpu
from jax.experimental.pallas import tpu_sc as plsc

S = 1024
R = 65536

_NC = 2
_NS = 16
_LANES = 16
_NW = _NC * _NS
_CHUNK = R // _NW


def _sc_rates_body(ab_hbm, idx0_hbm, idx1_hbm, al_hbm, be_hbm, ga_hbm,
                   ac_hbm, af_hbm, sc_hbm, out_hbm,
                   ab_v, i0_v, i1_v, al_v, be_v, ga_v, ac_v, af_v, sc_v, f_v):
    wid = lax.axis_index("s") * _NC + lax.axis_index("c")
    base = wid * _CHUNK
    pltpu.sync_copy(ab_hbm, ab_v)
    pltpu.sync_copy(sc_hbm, sc_v)
    pltpu.sync_copy(idx0_hbm.at[pl.ds(base, _CHUNK)], i0_v)
    pltpu.sync_copy(idx1_hbm.at[pl.ds(base, _CHUNK)], i1_v)
    pltpu.sync_copy(al_hbm.at[pl.ds(base, _CHUNK)], al_v)
    pltpu.sync_copy(be_hbm.at[pl.ds(base, _CHUNK)], be_v)
    pltpu.sync_copy(ga_hbm.at[pl.ds(base, _CHUNK)], ga_v)
    pltpu.sync_copy(ac_hbm.at[pl.ds(base, _CHUNK)], ac_v)
    pltpu.sync_copy(af_hbm.at[pl.ds(base, _CHUNK)], af_v)

    ltv = sc_v[pl.ds(0, _LANES)]
    niv = sc_v[pl.ds(16, _LANES)]
    crv = sc_v[pl.ds(32, _LANES)]
    fuvv = sc_v[pl.ds(48, _LANES)]

    def step(i, carry):
        off = i * _LANES
        sl = pl.ds(off, _LANES)
        a0 = plsc.load_gather(ab_v, [i0_v[sl]])
        a1 = plsc.load_gather(ab_v, [i1_v[sl]])
        rate = (al_v[sl] * jnp.exp(be_v[sl] * ltv + ga_v[sl] * niv)
                + ac_v[sl] * crv + af_v[sl] * fuvv)
        f_v[sl] = rate * a0 * a1
        return carry

    lax.fori_loop(0, _CHUNK // _LANES, step, 0, unroll=4)
    pltpu.sync_copy(f_v, out_hbm.at[pl.ds(base, _CHUNK)])


_sc_rates = pl.kernel(
    _sc_rates_body,
    out_type=jax.ShapeDtypeStruct((R,), jnp.float32),
    mesh=plsc.VectorSubcoreMesh(
        core_axis_name="c", subcore_axis_name="s", num_cores=_NC,
        num_subcores=_NS),
    scratch_types=[
        pltpu.VMEM((S,), jnp.float32),
        pltpu.VMEM((_CHUNK,), jnp.int32),
        pltpu.VMEM((_CHUNK,), jnp.int32),
        pltpu.VMEM((_CHUNK,), jnp.float32),
        pltpu.VMEM((_CHUNK,), jnp.float32),
        pltpu.VMEM((_CHUNK,), jnp.float32),
        pltpu.VMEM((_CHUNK,), jnp.float32),
        pltpu.VMEM((_CHUNK,), jnp.float32),
        pltpu.VMEM((64,), jnp.float32),
        pltpu.VMEM((_CHUNK,), jnp.float32),
    ],
    compiler_params=pltpu.CompilerParams(needs_layout_passes=False),
)


_SSC = 256
_STC = S - _SSC
_ROWS_W = _SSC // _NW
_CH = 4096

_SB = 64
_NSTEPS = _STC // _SB


def _sc_mv_body(inc_hbm, v_hbm, out_hbm, inc_v, v_v, st_v,
                sv0, sv1, si0, si1):
    wid = lax.axis_index("s") * _NC + lax.axis_index("c")
    row0 = _STC + wid * _ROWS_W
    nchunks = R // _CH
    svs = (sv0, sv1)
    sis = (si0, si1)

    def start(c, buf):
        off = c * _CH
        dv = pltpu.async_copy(v_hbm.at[pl.ds(off, _CH)], v_v.at[buf],
                              svs[buf])
        di = pltpu.async_copy(
            inc_hbm.at[pl.ds(row0, _ROWS_W), pl.ds(off, _CH)],
            inc_v.at[buf], sis[buf])
        return dv, di

    pend = start(0, 0)
    accs = (jnp.zeros((_LANES,), jnp.float32),) * _ROWS_W
    for c in range(nchunks):
        buf = c % 2
        pend[0].wait()
        pend[1].wait()
        if c + 1 < nchunks:
            pend = start(c + 1, 1 - buf)

        def jstep(j, a, buf=buf):
            sl = pl.ds(j * _LANES, _LANES)
            v16 = v_v[buf, sl]
            return tuple(a[k] + inc_v[buf, k, sl] * v16
                         for k in range(_ROWS_W))

        accs = lax.fori_loop(0, _CH // _LANES, jstep, accs, unroll=4)

    for k in range(_ROWS_W):
        st_v[pl.ds(k * _LANES, _LANES)] = accs[k]
    pltpu.sync_copy(st_v, out_hbm.at[pl.ds(wid * _ROWS_W * _LANES,
                                           _ROWS_W * _LANES)])


_sc_mv = pl.kernel(
    _sc_mv_body,
    out_type=jax.ShapeDtypeStruct((_SSC * _LANES,), jnp.float32),
    mesh=plsc.VectorSubcoreMesh(
        core_axis_name="c", subcore_axis_name="s", num_cores=_NC,
        num_subcores=_NS),
    scratch_types=[
        pltpu.VMEM((2, _ROWS_W, _CH), jnp.float32),
        pltpu.VMEM((2, _CH), jnp.float32),
        pltpu.VMEM((_ROWS_W * _LANES,), jnp.float32),
        pltpu.SemaphoreType.DMA,
        pltpu.SemaphoreType.DMA,
        pltpu.SemaphoreType.DMA,
        pltpu.SemaphoreType.DMA,
    ],
    compiler_params=pltpu.CompilerParams(needs_layout_passes=False),
)


def _tc_matvec_body(inc_ref, v_ref, out_ref):
    out_ref[...] = lax.dot_general(
        inc_ref[...], v_ref[...],
        dimension_numbers=(((1,), (1,)), ((), ())),
        preferred_element_type=jnp.float32)


_tc_matvec = pl.pallas_call(
    _tc_matvec_body,
    grid=(_NSTEPS,),
    in_specs=[
        pl.BlockSpec((_SB, R), lambda i: (i, 0)),
        pl.BlockSpec((1, R), lambda i: (0, 0)),
    ],
    out_specs=pl.BlockSpec((_SB, 1), lambda i: (i, 0)),
    out_shape=jax.ShapeDtypeStruct((_STC, 1), jnp.float32),
    compiler_params=pltpu.CompilerParams(
        dimension_semantics=("arbitrary",)),
)


def kernel(time, abundances, temperature, cr_rate, fuv_rate, incidence,
           alpha, beta, gamma, alpha_cr, alpha_fuv, species_idx):
    idx2 = species_idx.reshape(R, 2)
    idx0 = idx2[:, 0]
    idx1 = idx2[:, 1]

    scal = jnp.concatenate([
        jnp.full((16,), jnp.log(temperature / 300.0), dtype=jnp.float32),
        jnp.full((16,), -1.0 / temperature, dtype=jnp.float32),
        jnp.full((16,), cr_rate, dtype=jnp.float32),
        jnp.full((16,), fuv_rate, dtype=jnp.float32),
    ])

    v = _sc_rates(abundances, idx0, idx1, alpha, beta, gamma,
                  alpha_cr, alpha_fuv, scal)

    out_sc = _sc_mv(incidence, v)
    out_tc = _tc_matvec(incidence, v.reshape(1, R))
    return jnp.concatenate([
        out_tc.reshape(_STC),
        out_sc.reshape(_SSC, _LANES).sum(axis=1),
    ])


# speedup vs baseline: 1.1351x; 1.0913x over previous
import jax
import jax.numpy as jnp
from jax import lax
from jax.experimental import pallas as pl
from jax.experimental.pallas import tpu as pltpu
from jax.experimental.pallas import tpu_sc as plsc

S = 1024
R = 65536

_NC = 2
_NS = 16
_LANES = 16
_NW = _NC * _NS
_CHUNK = R // _NW


def _sc_factor_body(ab_hbm, idx0_hbm, idx1_hbm, out_hbm, ab_v, i0_v, i1_v, f_v):
    wid = lax.axis_index("s") * _NC + lax.axis_index("c")
    base = wid * _CHUNK
    pltpu.sync_copy(ab_hbm, ab_v)
    pltpu.sync_copy(idx0_hbm.at[pl.ds(base, _CHUNK)], i0_v)
    pltpu.sync_copy(idx1_hbm.at[pl.ds(base, _CHUNK)], i1_v)

    def step(i, carry):
        off = i * _LANES
        sl = pl.ds(off, _LANES)
        a0 = plsc.load_gather(ab_v, [i0_v[sl]])
        a1 = plsc.load_gather(ab_v, [i1_v[sl]])
        f_v[sl] = a0 * a1
        return carry

    lax.fori_loop(0, _CHUNK // _LANES, step, 0, unroll=4)
    pltpu.sync_copy(f_v, out_hbm.at[pl.ds(base, _CHUNK)])


_sc_factor = pl.kernel(
    _sc_factor_body,
    out_type=jax.ShapeDtypeStruct((R,), jnp.float32),
    mesh=plsc.VectorSubcoreMesh(
        core_axis_name="c", subcore_axis_name="s", num_cores=_NC,
        num_subcores=_NS),
    scratch_types=[
        pltpu.VMEM((S,), jnp.float32),
        pltpu.VMEM((_CHUNK,), jnp.int32),
        pltpu.VMEM((_CHUNK,), jnp.int32),
        pltpu.VMEM((_CHUNK,), jnp.float32),
    ],
    compiler_params=pltpu.CompilerParams(needs_layout_passes=False),
)


_RB = 2048
_KSTEPS = R // _RB


def _tc_matvec_body(s_ref, inc_ref, al_ref, be_ref, ga_ref, ac_ref, af_ref,
                    fa_ref, out_ref):
    i = pl.program_id(0)
    lt = s_ref[0, 0]
    ninvT = s_ref[0, 1]
    cr = s_ref[0, 2]
    fuv = s_ref[0, 3]
    rates = (al_ref[...] * jnp.exp(be_ref[...] * lt + ga_ref[...] * ninvT)
             + ac_ref[...] * cr + af_ref[...] * fuv)
    v = rates * fa_ref[...]

    @pl.when(i == 0)
    def _init():
        out_ref[...] = jnp.zeros_like(out_ref)

    out_ref[...] += lax.dot_general(
        inc_ref[...], v,
        dimension_numbers=(((1,), (1,)), ((), ())),
        preferred_element_type=jnp.float32)


def _vec_spec():
    return pl.BlockSpec((1, _RB), lambda i: (0, i))


_tc_matvec = pl.pallas_call(
    _tc_matvec_body,
    grid=(_KSTEPS,),
    in_specs=[
        pl.BlockSpec(memory_space=pltpu.SMEM),
        pl.BlockSpec((S, _RB), lambda i: (0, i)),
        _vec_spec(), _vec_spec(), _vec_spec(), _vec_spec(), _vec_spec(),
        _vec_spec(),
    ],
    out_specs=pl.BlockSpec((S, 1), lambda i: (0, 0)),
    out_shape=jax.ShapeDtypeStruct((S, 1), jnp.float32),
    compiler_params=pltpu.CompilerParams(
        dimension_semantics=("arbitrary",)),
)


def kernel(time, abundances, temperature, cr_rate, fuv_rate, incidence,
           alpha, beta, gamma, alpha_cr, alpha_fuv, species_idx):
    idx2 = species_idx.reshape(R, 2)
    idx0 = idx2[:, 0]
    idx1 = idx2[:, 1]

    factor = _sc_factor(abundances, idx0, idx1)

    scal = jnp.stack([
        jnp.log(temperature / 300.0),
        -1.0 / temperature,
        cr_rate,
        fuv_rate,
    ]).reshape(1, 4)

    out = _tc_matvec(
        scal, incidence,
        alpha.reshape(1, R), beta.reshape(1, R), gamma.reshape(1, R),
        alpha_cr.reshape(1, R), alpha_fuv.reshape(1, R),
        factor.reshape(1, R))
    return out.reshape(S)


# speedup vs baseline: 1.1427x; 1.0067x over previous
import jax
import jax.numpy as jnp
from jax import lax
from jax.experimental import pallas as pl
from jax.experimental.pallas import tpu as pltpu
from jax.experimental.pallas import tpu_sc as plsc

S = 1024
R = 65536

_NC = 2
_NS = 16
_LANES = 16
_NW = _NC * _NS
_CHUNK = R // _NW


def _sc_factor_body(ab_hbm, idx0_hbm, idx1_hbm, out_hbm, ab_v, i0_v, i1_v, f_v):
    wid = lax.axis_index("s") * _NC + lax.axis_index("c")
    base = wid * _CHUNK
    pltpu.sync_copy(ab_hbm, ab_v)
    pltpu.sync_copy(idx0_hbm.at[pl.ds(base, _CHUNK)], i0_v)
    pltpu.sync_copy(idx1_hbm.at[pl.ds(base, _CHUNK)], i1_v)

    def step(i, carry):
        off = i * _LANES
        sl = pl.ds(off, _LANES)
        a0 = plsc.load_gather(ab_v, [i0_v[sl]])
        a1 = plsc.load_gather(ab_v, [i1_v[sl]])
        f_v[sl] = a0 * a1
        return carry

    lax.fori_loop(0, _CHUNK // _LANES, step, 0, unroll=4)
    pltpu.sync_copy(f_v, out_hbm.at[pl.ds(base, _CHUNK)])


_sc_factor = pl.kernel(
    _sc_factor_body,
    out_type=jax.ShapeDtypeStruct((R,), jnp.float32),
    mesh=plsc.VectorSubcoreMesh(
        core_axis_name="c", subcore_axis_name="s", num_cores=_NC,
        num_subcores=_NS),
    scratch_types=[
        pltpu.VMEM((S,), jnp.float32),
        pltpu.VMEM((_CHUNK,), jnp.int32),
        pltpu.VMEM((_CHUNK,), jnp.int32),
        pltpu.VMEM((_CHUNK,), jnp.float32),
    ],
    compiler_params=pltpu.CompilerParams(needs_layout_passes=False),
)


_RB = 4096
_KSTEPS = R // _RB


def _tc_matvec_body(s_ref, inc_ref, al_ref, be_ref, ga_ref, ac_ref, af_ref,
                    fa_ref, out_ref):
    i = pl.program_id(0)
    lt = s_ref[0, 0]
    ninvT = s_ref[0, 1]
    cr = s_ref[0, 2]
    fuv = s_ref[0, 3]
    rates = (al_ref[...] * jnp.exp(be_ref[...] * lt + ga_ref[...] * ninvT)
             + ac_ref[...] * cr + af_ref[...] * fuv)
    v = rates * fa_ref[...]

    @pl.when(i == 0)
    def _init():
        out_ref[...] = jnp.zeros_like(out_ref)

    out_ref[...] += lax.dot_general(
        inc_ref[...], v,
        dimension_numbers=(((1,), (1,)), ((), ())),
        preferred_element_type=jnp.float32)


def _vec_spec():
    return pl.BlockSpec((1, _RB), lambda i: (0, i))


_tc_matvec = pl.pallas_call(
    _tc_matvec_body,
    grid=(_KSTEPS,),
    in_specs=[
        pl.BlockSpec(memory_space=pltpu.SMEM),
        pl.BlockSpec((S, _RB), lambda i: (0, i)),
        _vec_spec(), _vec_spec(), _vec_spec(), _vec_spec(), _vec_spec(),
        _vec_spec(),
    ],
    out_specs=pl.BlockSpec((S, 1), lambda i: (0, 0)),
    out_shape=jax.ShapeDtypeStruct((S, 1), jnp.float32),
    compiler_params=pltpu.CompilerParams(
        dimension_semantics=("arbitrary",)),
)


def kernel(time, abundances, temperature, cr_rate, fuv_rate, incidence,
           alpha, beta, gamma, alpha_cr, alpha_fuv, species_idx):
    idx2 = species_idx.reshape(R, 2)
    idx0 = idx2[:, 0]
    idx1 = idx2[:, 1]

    factor = _sc_factor(abundances, idx0, idx1)

    scal = jnp.stack([
        jnp.log(temperature / 300.0),
        -1.0 / temperature,
        cr_rate,
        fuv_rate,
    ]).reshape(1, 4)

    out = _tc_matvec(
        scal, incidence,
        alpha.reshape(1, R), beta.reshape(1, R), gamma.reshape(1, R),
        alpha_cr.reshape(1, R), alpha_fuv.reshape(1, R),
        factor.reshape(1, R))
    return out.reshape(S)
